# baseline (speedup 1.0000x reference)
import functools

import jax
import jax.numpy as jnp
from jax import lax
from jax.experimental import pallas as pl
from jax.experimental.pallas import tpu as pltpu
from jax.experimental.pallas import tpu_sc as plsc

BATCH = 4096
HIST = 50
HPAD = 56
D = 128
NW = 32
BPW = BATCH // NW

_mesh = plsc.VectorSubcoreMesh(core_axis_name="c", subcore_axis_name="s")


@functools.partial(
    pl.kernel,
    mesh=_mesh,
    out_type=jax.ShapeDtypeStruct((BATCH, HIST, D), jnp.float32),
    scratch_types=[
        pltpu.VMEM((BPW * HPAD,), jnp.int32),
        pltpu.VMEM((HIST, D), jnp.float32),
        pltpu.VMEM((HIST, D), jnp.float32),
        pltpu.SemaphoreType.DMA,
        pltpu.SemaphoreType.DMA,
    ],
    compiler_params=pltpu.CompilerParams(use_tc_tiling_on_sc=True),
)
def _gather_kernel(idx_hbm, table_hbm, out_hbm, idx_v, buf0, buf1, g0, g1):
    wid = lax.axis_index("s") * 2 + lax.axis_index("c")
    b0 = wid * BPW
    pltpu.sync_copy(idx_hbm.at[pl.ds(b0 * HPAD, BPW * HPAD)], idx_v)

    def gather(m, buf, sem):
        return pltpu.make_async_copy(
            table_hbm.at[idx_v.at[pl.ds(m * HPAD, HIST)]], buf, sem
        )

    def writeback(m, buf):
        pltpu.sync_copy(buf, out_hbm.at[b0 + m])

    gather(0, buf0, g0).start()

    def body(i, carry):
        m0 = 2 * i
        gather(m0 + 1, buf1, g1).start()
        gather(m0, buf0, g0).wait()
        writeback(m0, buf0)

        @pl.when(i < BPW // 2 - 1)
        def _():
            gather(m0 + 2, buf0, g0).start()

        gather(m0 + 1, buf1, g1).wait()
        writeback(m0 + 1, buf1)
        return carry

    lax.fori_loop(0, BPW // 2, body, 0)


def kernel(input, weight):
    idx = jnp.pad(input, ((0, 0), (0, HPAD - HIST))).reshape(-1)
    return _gather_kernel(idx, weight)


# speedup vs baseline: 2.0448x; 2.0448x over previous
import functools

import jax
import jax.numpy as jnp
from jax import lax
from jax.experimental import pallas as pl
from jax.experimental.pallas import tpu as pltpu
from jax.experimental.pallas import tpu_sc as plsc

BATCH = 4096
HIST = 50
D = 128
NW = 32
BPW = BATCH // NW

_mesh = plsc.VectorSubcoreMesh(core_axis_name="c", subcore_axis_name="s")


@functools.partial(
    pl.kernel,
    mesh=_mesh,
    out_type=jax.ShapeDtypeStruct((HIST, BATCH, D), jnp.float32),
    scratch_types=[
        pltpu.VMEM((HIST, BPW), jnp.int32),
        pltpu.VMEM((BPW, D), jnp.float32),
        pltpu.VMEM((BPW, D), jnp.float32),
        pltpu.SemaphoreType.DMA,
        pltpu.SemaphoreType.DMA,
    ],
)
def _gather_kernel(idx_hbm, table_hbm, out_hbm, idx_v, buf0, buf1, g0, g1):
    wid = lax.axis_index("s") * 2 + lax.axis_index("c")
    b0 = wid * BPW
    pltpu.sync_copy(idx_hbm.at[pl.ds(0, HIST), pl.ds(b0, BPW)], idx_v)

    def gather(h, buf, sem):
        return pltpu.make_async_copy(table_hbm.at[idx_v.at[h]], buf, sem)

    def writeback(h, buf):
        pltpu.sync_copy(buf, out_hbm.at[h, pl.ds(b0, BPW)])

    gather(0, buf0, g0).start()

    def body(i, carry):
        h0 = 2 * i
        gather(h0 + 1, buf1, g1).start()
        gather(h0, buf0, g0).wait()
        writeback(h0, buf0)

        @pl.when(i < HIST // 2 - 1)
        def _():
            gather(h0 + 2, buf0, g0).start()

        gather(h0 + 1, buf1, g1).wait()
        writeback(h0 + 1, buf1)
        return carry

    lax.fori_loop(0, HIST // 2, body, 0)


def kernel(input, weight):
    idx_t = jnp.transpose(input)
    out = _gather_kernel(idx_t, weight)
    return jnp.transpose(out, (1, 0, 2))
